# Initial kernel scaffold; baseline (speedup 1.0000x reference)
import jax
import jax.numpy as jnp
from jax.experimental import pallas as pl


def kernel(x, edge_index, W1_l, b1_l, W1_r, b1_r, W2_l, b2_l, W2_r, b2_r):
    raise NotImplementedError("write your pallas kernel here")


# speedup vs baseline: 4.5722x; 4.5722x over previous
import functools

import jax
import jax.numpy as jnp
from jax import lax
from jax.experimental import pallas as pl
from jax.experimental.pallas import tpu as pltpu
from jax.experimental.pallas import tpu_sc as plsc

N_NODES = 10000
D = 128
N_EDGES = 320000
NC = 2
NS = 16
NW = NC * NS
E_PER_W = N_EDGES // NW
CHUNK = 64
E_PAD_W = 10240
NCHUNK = E_PAD_W // CHUNK
ACC_ROWS = N_NODES + 16
NR = 4
NQ = 2 * NR
GDEPTH = NR - 2
IAHEAD = GDEPTH + 2
ROWS_A = 624
ROWS_LAST = N_NODES - (NS - 1) * ROWS_A
ZROWS = 16

_mesh = plsc.VectorSubcoreMesh(core_axis_name="c", subcore_axis_name="s")


def _zero_fill(ref, nrows, width):
  zvec = jnp.zeros((16,), jnp.float32)
  per_row = width // 16

  def zf(i, _):
    ref[i // per_row, pl.ds((i % per_row) * 16, 16)] = zvec
    return 0

  lax.fori_loop(0, nrows * per_row, zf, 0)


def _drain(sid, sh_ref, out_ref):
  row0 = sid * ROWS_A

  @pl.when(sid < NS - 1)
  def _():
    pltpu.sync_copy(sh_ref.at[pl.ds(row0, ROWS_A)],
                    out_ref.at[pl.ds(row0, ROWS_A)])

  @pl.when(sid == NS - 1)
  def _():
    pltpu.sync_copy(sh_ref.at[pl.ds(row0, ROWS_LAST)],
                    out_ref.at[pl.ds(row0, ROWS_LAST)])


def _sc_agg_body(feat, src_hbm, dst_hbm, *rest):
  (acc_out, sidx_v, didx_v, rows_v, zero_v, acc_sh) = rest[:6]
  sems = rest[6:]
  sis = sems[0:NQ]
  sdi = sems[NQ:2 * NQ]
  sg = sems[2 * NQ:2 * NQ + NR]
  ss = sems[2 * NQ + NR:2 * NQ + 2 * NR]

  cid = lax.axis_index("c")
  sid = lax.axis_index("s")
  wid = sid * NC + cid
  row0 = sid * ROWS_A
  nz = jnp.where(sid == NS - 1, ROWS_LAST // ZROWS, ROWS_A // ZROWS)

  _zero_fill(zero_v, ZROWS, D)

  def zcopy(kk, _):
    pltpu.sync_copy(zero_v, acc_sh.at[pl.ds(row0 + kk * ZROWS, ZROWS)])
    return 0

  lax.fori_loop(0, nz, zcopy, 0)

  plsc.subcore_barrier()

  def idx_copy(i, q):
    pltpu.async_copy(src_hbm.at[wid, i], sidx_v.at[q], sis[q])
    pltpu.async_copy(dst_hbm.at[wid, i], didx_v.at[q], sdi[q])

  def wait_sidx(i, q):
    pltpu.make_async_copy(src_hbm.at[wid, i], sidx_v.at[q], sis[q]).wait()

  def wait_didx(i, q):
    pltpu.make_async_copy(dst_hbm.at[wid, i], didx_v.at[q], sdi[q]).wait()

  def gather(q, b):
    pltpu.async_copy(feat.at[sidx_v.at[q]], rows_v.at[b], sg[b])

  def wait_gather(q, b):
    pltpu.make_async_copy(
        feat.at[sidx_v.at[q]], rows_v.at[b], sg[b]).wait()

  def scatter(q, b):
    pltpu.async_copy(rows_v.at[b], acc_sh.at[didx_v.at[q]], ss[b], add=True)

  def wait_scatter(q, b):
    pltpu.make_async_copy(
        rows_v.at[b], acc_sh.at[didx_v.at[q]], ss[b]).wait()

  for c in range(IAHEAD):
    idx_copy(c, c)
  for c in range(GDEPTH):
    wait_sidx(c, c)
    gather(c, c % NR)

  def step(i, j):
    q = j
    r = j % NR

    @pl.when(i + GDEPTH < NCHUNK)
    def _():
      wait_sidx(i + GDEPTH, (j + GDEPTH) % NQ)

      @pl.when(i >= 2)
      def _():
        wait_scatter((j - 2) % NQ, (j - 2) % NR)

      gather((j + GDEPTH) % NQ, (j + GDEPTH) % NR)

    wait_gather(q, r)
    wait_didx(i, q)
    scatter(q, r)

    @pl.when(i + IAHEAD < NCHUNK)
    def _():
      idx_copy(i + IAHEAD, (j + IAHEAD) % NQ)

  def loop_body(k, _):
    for j in range(NQ):
      i = NQ * k + j

      @pl.when(i < NCHUNK)
      def _():
        step(i, j)

    return 0

  lax.fori_loop(0, (NCHUNK + NQ - 1) // NQ, loop_body, 0)

  for t in range(NCHUNK - NR, NCHUNK):
    wait_scatter(t % NQ, t % NR)

  plsc.subcore_barrier()
  _drain(sid, acc_sh, acc_out.at[cid])


_sc_agg = pl.kernel(
    _sc_agg_body,
    out_type=jax.ShapeDtypeStruct((NC, N_NODES, D), jnp.float32),
    mesh=_mesh,
    scratch_types=[
        pltpu.VMEM((NQ, CHUNK), jnp.int32),
        pltpu.VMEM((NQ, CHUNK), jnp.int32),
        pltpu.VMEM((NR, CHUNK, D), jnp.float32),
        pltpu.VMEM((ZROWS, D), jnp.float32),
        pltpu.VMEM_SHARED((ACC_ROWS, D), jnp.float32),
    ] + [pltpu.SemaphoreType.DMA] * (2 * NQ + 2 * NR),
)


def _sc_deg_body(dst_hbm, deg_out, didx_v, hist_v):
  cid = lax.axis_index("c")
  sid = lax.axis_index("s")
  wid = sid * NC + cid

  zvec = jnp.zeros((16,), jnp.float32)

  def hz(i, _):
    hist_v[pl.ds(i * 16, 16)] = zvec
    return 0

  lax.fori_loop(0, N_NODES // 16, hz, 0)

  pltpu.sync_copy(dst_hbm.at[wid], didx_v)

  def cnt(j, _):
    vals = didx_v[pl.ds(j * 16, 16)]
    ones_d = vals.astype(jnp.float32) * 0.0 + 1.0
    plsc.addupdate_scatter(hist_v, [vals], ones_d)
    return 0

  lax.fori_loop(0, E_PER_W // 16, cnt, 0)

  pltpu.sync_copy(hist_v, deg_out.at[cid, sid])


_sc_deg = pl.kernel(
    _sc_deg_body,
    out_type=jax.ShapeDtypeStruct((NC, NS, N_NODES), jnp.float32),
    mesh=_mesh,
    scratch_types=[
        pltpu.VMEM((E_PER_W,), jnp.int32),
        pltpu.VMEM((N_NODES,), jnp.float32),
    ],
    compiler_params=pltpu.CompilerParams(needs_layout_passes=False),
)


def _tc_layer(p_ref, degp_ref, x_ref, wl_ref, wr_ref, bl_ref, br_ref, o_ref,
              *, relu):
  agg = p_ref[0] + p_ref[1]
  deg = jnp.sum(degp_ref[...], axis=1, keepdims=True)
  mean = agg / jnp.maximum(deg, 1.0)
  h = (jnp.dot(mean, wl_ref[...], preferred_element_type=jnp.float32)
       + jnp.dot(x_ref[...], wr_ref[...], preferred_element_type=jnp.float32)
       + bl_ref[...] + br_ref[...])
  o_ref[...] = jnp.maximum(h, 0.0) if relu else h


_TC_BLK = 2000


def _tc_call(p, degp, x, wl, wr, bl, br, relu):
  return pl.pallas_call(
      functools.partial(_tc_layer, relu=relu),
      grid=(N_NODES // _TC_BLK,),
      in_specs=[
          pl.BlockSpec((NC, _TC_BLK, D), lambda i: (0, i, 0)),
          pl.BlockSpec((_TC_BLK, NW), lambda i: (i, 0)),
          pl.BlockSpec((_TC_BLK, D), lambda i: (i, 0)),
          pl.BlockSpec((D, D), lambda i: (0, 0)),
          pl.BlockSpec((D, D), lambda i: (0, 0)),
          pl.BlockSpec((1, D), lambda i: (0, 0)),
          pl.BlockSpec((1, D), lambda i: (0, 0)),
      ],
      out_specs=pl.BlockSpec((_TC_BLK, D), lambda i: (i, 0)),
      out_shape=jax.ShapeDtypeStruct((N_NODES, D), jnp.float32),
  )(p, degp, x, wl, wr, bl, br)


def kernel(x, edge_index, W1_l, b1_l, W1_r, b1_r, W2_l, b2_l, W2_r, b2_r):
  npad = E_PAD_W - E_PER_W
  src = jnp.concatenate(
      [edge_index[0].astype(jnp.int32).reshape(NW, E_PER_W),
       jnp.zeros((NW, npad), jnp.int32)], axis=1).reshape(NW, NCHUNK, CHUNK)
  dst = jnp.concatenate(
      [edge_index[1].astype(jnp.int32).reshape(NW, E_PER_W),
       jnp.full((NW, npad), N_NODES, jnp.int32)],
      axis=1).reshape(NW, NCHUNK, CHUNK)

  dst_flat = edge_index[1].astype(jnp.int32).reshape(NW, E_PER_W)
  degp = _sc_deg(dst_flat)
  degp = degp.reshape(NW, N_NODES).T
  p1 = _sc_agg(x, src, dst)
  h = _tc_call(p1, degp, x, W1_l, W1_r,
               b1_l.reshape(1, D), b1_r.reshape(1, D), relu=True)
  p2 = _sc_agg(h, src, dst)
  return _tc_call(p2, degp, h, W2_l, W2_r,
                  b2_l.reshape(1, D), b2_r.reshape(1, D), relu=False)


# speedup vs baseline: 13.1645x; 2.8792x over previous
import functools

import jax
import jax.numpy as jnp
from jax import lax
from jax.experimental import pallas as pl
from jax.experimental.pallas import tpu as pltpu
from jax.experimental.pallas import tpu_sc as plsc

N_NODES = 10000
D = 128
N_EDGES = 320000
NC = 2
NS = 16
NW = NC * NS
E_PER_W = N_EDGES // NW
CHUNK = 40
NCHUNK = E_PER_W // CHUNK
ACC_ROWS = N_NODES
NR = 4
NQ = 2 * NR
GDEPTH = NR - 2
IAHEAD = GDEPTH + 2
ROWS_A = 624
ROWS_LAST = N_NODES - (NS - 1) * ROWS_A
ZROWS = 16

_mesh = plsc.VectorSubcoreMesh(core_axis_name="c", subcore_axis_name="s")


def _zero_fill(ref, nrows, width):
  zvec = jnp.zeros((16,), jnp.float32)
  per_row = width // 16

  def zf(i, _):
    ref[i // per_row, pl.ds((i % per_row) * 16, 16)] = zvec
    return 0

  lax.fori_loop(0, nrows * per_row, zf, 0)


def _drain(sid, sh_ref, out_ref):
  row0 = sid * ROWS_A

  @pl.when(sid < NS - 1)
  def _():
    pltpu.sync_copy(sh_ref.at[pl.ds(row0, ROWS_A)],
                    out_ref.at[pl.ds(row0, ROWS_A)])

  @pl.when(sid == NS - 1)
  def _():
    pltpu.sync_copy(sh_ref.at[pl.ds(row0, ROWS_LAST)],
                    out_ref.at[pl.ds(row0, ROWS_LAST)])


def _sc_agg_body(feat, src_hbm, dst_hbm, *rest):
  (acc_out, sidx_v, didx_v, rows_v, zero_v, acc_sh) = rest[:6]
  sems = rest[6:]
  sis = sems[0:NQ]
  sdi = sems[NQ:2 * NQ]
  sg = sems[2 * NQ:2 * NQ + NR]
  ss = sems[2 * NQ + NR:2 * NQ + 2 * NR]

  cid = lax.axis_index("c")
  sid = lax.axis_index("s")
  wid = sid * NC + cid
  row0 = sid * ROWS_A
  nz = jnp.where(sid == NS - 1, ROWS_LAST // ZROWS, ROWS_A // ZROWS)

  _zero_fill(zero_v, ZROWS, D)

  def zcopy(kk, _):
    pltpu.sync_copy(zero_v, acc_sh.at[pl.ds(row0 + kk * ZROWS, ZROWS)])
    return 0

  lax.fori_loop(0, nz, zcopy, 0)

  plsc.subcore_barrier()

  def idx_copy(i, q):
    pltpu.async_copy(src_hbm.at[wid, i], sidx_v.at[q], sis[q])
    pltpu.async_copy(dst_hbm.at[wid, i], didx_v.at[q], sdi[q])

  def wait_sidx(i, q):
    pltpu.make_async_copy(src_hbm.at[wid, i], sidx_v.at[q], sis[q]).wait()

  def wait_didx(i, q):
    pltpu.make_async_copy(dst_hbm.at[wid, i], didx_v.at[q], sdi[q]).wait()

  def gather(q, b):
    pltpu.async_copy(feat.at[sidx_v.at[q]], rows_v.at[b], sg[b])

  def wait_gather(q, b):
    pltpu.make_async_copy(
        feat.at[sidx_v.at[q]], rows_v.at[b], sg[b]).wait()

  def scatter(q, b):
    pltpu.async_copy(rows_v.at[b], acc_sh.at[didx_v.at[q]], ss[b], add=True)

  def wait_scatter(q, b):
    pltpu.make_async_copy(
        rows_v.at[b], acc_sh.at[didx_v.at[q]], ss[b]).wait()

  for c in range(IAHEAD):
    idx_copy(c, c)
  for c in range(GDEPTH):
    wait_sidx(c, c)
    gather(c, c % NR)

  def step(i, j):
    q = j
    r = j % NR

    @pl.when(i + GDEPTH < NCHUNK)
    def _():
      wait_sidx(i + GDEPTH, (j + GDEPTH) % NQ)

      @pl.when(i >= 2)
      def _():
        wait_scatter((j - 2) % NQ, (j - 2) % NR)

      gather((j + GDEPTH) % NQ, (j + GDEPTH) % NR)

    wait_gather(q, r)
    wait_didx(i, q)
    scatter(q, r)

    @pl.when(i + IAHEAD < NCHUNK)
    def _():
      idx_copy(i + IAHEAD, (j + IAHEAD) % NQ)

  def loop_body(k, _):
    for j in range(NQ):
      i = NQ * k + j

      @pl.when(i < NCHUNK)
      def _():
        step(i, j)

    return 0

  lax.fori_loop(0, (NCHUNK + NQ - 1) // NQ, loop_body, 0)

  for t in range(NCHUNK - NR, NCHUNK):
    wait_scatter(t % NQ, t % NR)

  plsc.subcore_barrier()
  _drain(sid, acc_sh, acc_out.at[cid])


_sc_agg = pl.kernel(
    _sc_agg_body,
    out_type=jax.ShapeDtypeStruct((NC, N_NODES, D), jnp.float32),
    mesh=_mesh,
    scratch_types=[
        pltpu.VMEM((NQ, CHUNK), jnp.int32),
        pltpu.VMEM((NQ, CHUNK), jnp.int32),
        pltpu.VMEM((NR, CHUNK, D), jnp.float32),
        pltpu.VMEM((ZROWS, D), jnp.float32),
        pltpu.VMEM_SHARED((ACC_ROWS, D), jnp.float32),
    ] + [pltpu.SemaphoreType.DMA] * (2 * NQ + 2 * NR),
)


def _sc_deg_body(dst_hbm, deg_out, didx_v, hist_v):
  cid = lax.axis_index("c")
  sid = lax.axis_index("s")
  wid = sid * NC + cid

  zvec = jnp.zeros((16,), jnp.float32)

  def hz(i, _):
    hist_v[pl.ds(i * 16, 16)] = zvec
    return 0

  lax.fori_loop(0, N_NODES // 16, hz, 0)

  pltpu.sync_copy(dst_hbm.at[wid], didx_v)

  def cnt(j, _):
    vals = didx_v[pl.ds(j * 16, 16)]
    ones_d = vals.astype(jnp.float32) * 0.0 + 1.0
    plsc.addupdate_scatter(hist_v, [vals], ones_d)
    return 0

  lax.fori_loop(0, E_PER_W // 16, cnt, 0)

  pltpu.sync_copy(hist_v, deg_out.at[cid, sid])


_sc_deg = pl.kernel(
    _sc_deg_body,
    out_type=jax.ShapeDtypeStruct((NC, NS, N_NODES), jnp.float32),
    mesh=_mesh,
    scratch_types=[
        pltpu.VMEM((E_PER_W,), jnp.int32),
        pltpu.VMEM((N_NODES,), jnp.float32),
    ],
    compiler_params=pltpu.CompilerParams(needs_layout_passes=False),
)


def _tc_layer(p_ref, degp_ref, x_ref, wl_ref, wr_ref, bl_ref, br_ref, o_ref,
              *, relu):
  agg = p_ref[0] + p_ref[1]
  deg = jnp.sum(degp_ref[...], axis=1, keepdims=True)
  mean = agg / jnp.maximum(deg, 1.0)
  h = (jnp.dot(mean, wl_ref[...], preferred_element_type=jnp.float32)
       + jnp.dot(x_ref[...], wr_ref[...], preferred_element_type=jnp.float32)
       + bl_ref[...] + br_ref[...])
  o_ref[...] = jnp.maximum(h, 0.0) if relu else h


_TC_BLK = 2000


def _tc_call(p, degp, x, wl, wr, bl, br, relu):
  return pl.pallas_call(
      functools.partial(_tc_layer, relu=relu),
      grid=(N_NODES // _TC_BLK,),
      in_specs=[
          pl.BlockSpec((NC, _TC_BLK, D), lambda i: (0, i, 0)),
          pl.BlockSpec((_TC_BLK, NW), lambda i: (i, 0)),
          pl.BlockSpec((_TC_BLK, D), lambda i: (i, 0)),
          pl.BlockSpec((D, D), lambda i: (0, 0)),
          pl.BlockSpec((D, D), lambda i: (0, 0)),
          pl.BlockSpec((1, D), lambda i: (0, 0)),
          pl.BlockSpec((1, D), lambda i: (0, 0)),
      ],
      out_specs=pl.BlockSpec((_TC_BLK, D), lambda i: (i, 0)),
      out_shape=jax.ShapeDtypeStruct((N_NODES, D), jnp.float32),
  )(p, degp, x, wl, wr, bl, br)


def kernel(x, edge_index, W1_l, b1_l, W1_r, b1_r, W2_l, b2_l, W2_r, b2_r):
  src = edge_index[0].astype(jnp.int32).reshape(NW, NCHUNK, CHUNK)
  dst = edge_index[1].astype(jnp.int32).reshape(NW, NCHUNK, CHUNK)

  dst_flat = edge_index[1].astype(jnp.int32).reshape(NW, E_PER_W)
  degp = _sc_deg(dst_flat)
  degp = degp.reshape(NW, N_NODES).T
  p1 = _sc_agg(x, src, dst)
  h = _tc_call(p1, degp, x, W1_l, W1_r,
               b1_l.reshape(1, D), b1_r.reshape(1, D), relu=True)
  p2 = _sc_agg(h, src, dst)
  return _tc_call(p2, degp, h, W2_l, W2_r,
                  b2_l.reshape(1, D), b2_r.reshape(1, D), relu=False)
